# baseline (speedup 1.0000x reference)
import dataclasses
import functools

import jax
import jax.numpy as jnp
import numpy as np
from jax import lax
from jax.experimental import pallas as pl
from jax.experimental.pallas import tpu as pltpu
from jax.experimental.pallas import tpu_sc as plsc

N = 10000
D = 128
NC = 2
NS = 16
LANES = 16
NW = NC * NS

NPAD = 10240
CH = 80
EPAD = NW * NPAD
CPW = EPAD // NW // CH
SLABC = 16
RPT = NPAD // NS

_mesh = plsc.VectorSubcoreMesh(
    core_axis_name="c", subcore_axis_name="s", num_cores=NC, num_subcores=NS
)

_sc_params = pltpu.CompilerParams()
if "needs_layout_passes" in pltpu.CompilerParams.__dataclass_fields__:
  _sc_params = dataclasses.replace(_sc_params, needs_layout_passes=False)


def _sc_hist(dst1d):
  @functools.partial(
      pl.kernel,
      out_type=jax.ShapeDtypeStruct((NW, NPAD), jnp.float32),
      mesh=_mesh,
      compiler_params=_sc_params,
      scratch_types=[
          pltpu.VMEM((CPW * CH,), jnp.int32),
          pltpu.VMEM((NPAD,), jnp.float32),
      ],
  )
  def k(d_hbm, o_hbm, didx, hist):
    c = lax.axis_index("c")
    s = lax.axis_index("s")
    wid = s * NC + c

    @pl.loop(0, NPAD, step=LANES)
    def _(i):
      hist.at[pl.ds(i, LANES)][...] = jnp.zeros((LANES,), jnp.float32)

    pltpu.sync_copy(d_hbm.at[pl.ds(wid * CPW * CH, CPW * CH)], didx)

    @pl.loop(0, CPW * CH, step=LANES)
    def _(t):
      idxv = didx.at[pl.ds(t, LANES)][...]
      plsc.addupdate_scatter(hist, [idxv], jnp.ones((LANES,), jnp.float32))

    pltpu.sync_copy(hist, o_hbm.at[wid])

  return k(dst1d)


def _sc_prop(g, src2, dst2):
  @functools.partial(
      pl.kernel,
      out_type=jax.ShapeDtypeStruct((NC, NPAD, D), jnp.float32),
      mesh=_mesh,
      compiler_params=_sc_params,
      scratch_types=[
          pltpu.VMEM((SLABC, CH), jnp.int32),
          pltpu.VMEM((SLABC, CH), jnp.int32),
          pltpu.VMEM((CH, D), jnp.float32),
          pltpu.VMEM((CH, D), jnp.float32),
          pltpu.VMEM_SHARED((NPAD, D), jnp.float32),
          pltpu.SemaphoreType.DMA,
          pltpu.SemaphoreType.DMA,
          pltpu.SemaphoreType.DMA,
      ],
  )
  def k(g_hbm, s_hbm, d_hbm, o_hbm, sidx, didx, rows0, rows1, acc,
        sem0, sem1, zsem):
    c = lax.axis_index("c")
    s = lax.axis_index("s")
    wid = c * NS + s
    base = wid * CPW

    pltpu.async_copy(s_hbm.at[pl.ds(base, SLABC)], sidx, sem0)
    pltpu.async_copy(d_hbm.at[pl.ds(base, SLABC)], didx, sem1)

    @pl.loop(0, CH)
    def _(r):
      @pl.loop(0, D, step=LANES)
      def _(l):
        rows0.at[r, pl.ds(l, LANES)][...] = jnp.zeros((LANES,), jnp.float32)

    @pl.loop(0, RPT, step=CH)
    def _(r0):
      pltpu.async_copy(rows0, acc.at[pl.ds(s * RPT + r0, CH)], zsem)

    @pl.loop(0, RPT, step=CH)
    def _(r0):
      pltpu.make_async_copy(
          rows0, acc.at[pl.ds(s * RPT + r0, CH)], zsem).wait()

    plsc.subcore_barrier()

    @pl.loop(0, CPW // SLABC)
    def _(sl):
      sbase = base + sl * SLABC

      @pl.when(sl > 0)
      def _():
        pltpu.async_copy(s_hbm.at[pl.ds(sbase, SLABC)], sidx, sem0)
        pltpu.async_copy(d_hbm.at[pl.ds(sbase, SLABC)], didx, sem1)

      pltpu.make_async_copy(s_hbm.at[pl.ds(sbase, SLABC)], sidx, sem0).wait()
      pltpu.make_async_copy(d_hbm.at[pl.ds(sbase, SLABC)], didx, sem1).wait()

      pltpu.async_copy(g_hbm.at[sidx.at[0]], rows0, sem0)

      @pl.loop(0, SLABC, step=2)
      def _(j):
        pltpu.async_copy(g_hbm.at[sidx.at[j + 1]], rows1, sem1)
        pltpu.make_async_copy(g_hbm.at[sidx.at[j]], rows0, sem0).wait()
        pltpu.sync_copy(rows0, acc.at[didx.at[j]], add=True)

        @pl.when(j + 2 < SLABC)
        def _():
          pltpu.async_copy(g_hbm.at[sidx.at[j + 2]], rows0, sem0)

        pltpu.make_async_copy(g_hbm.at[sidx.at[j + 1]], rows1, sem1).wait()
        pltpu.sync_copy(rows1, acc.at[didx.at[j + 1]], add=True)

    plsc.subcore_barrier()

    @pl.loop(0, RPT, step=CH)
    def _(r0):
      pltpu.async_copy(
          acc.at[pl.ds(s * RPT + r0, CH)],
          o_hbm.at[c, pl.ds(s * RPT + r0, CH)],
          zsem,
      )

    @pl.loop(0, RPT, step=CH)
    def _(r0):
      pltpu.make_async_copy(
          acc.at[pl.ds(s * RPT + r0, CH)],
          o_hbm.at[c, pl.ds(s * RPT + r0, CH)],
          zsem,
      ).wait()

  return k(g, src2, dst2)


_BR = 1024
_BN = 1000


def _tc_dinv(deg_p):
  def body(dp_ref, o_ref):
    deg = jnp.sum(dp_ref[...], axis=0) + 1.0
    o_ref[...] = lax.rsqrt(deg)

  return pl.pallas_call(
      body,
      out_shape=jax.ShapeDtypeStruct((NPAD // 128, 128), jnp.float32),
  )(deg_p)


def _tc_mm(x, W):
  def body(x_ref, w_ref, o_ref):
    o_ref[...] = jnp.dot(
        x_ref[...], w_ref[...], preferred_element_type=jnp.float32)

  return pl.pallas_call(
      body,
      grid=(N // _BN,),
      in_specs=[
          pl.BlockSpec((_BN, D), lambda i: (i, 0)),
          pl.BlockSpec((D, D), lambda i: (0, 0)),
      ],
      out_specs=pl.BlockSpec((_BN, D), lambda i: (i, 0)),
      out_shape=jax.ShapeDtypeStruct((NPAD, D), jnp.float32),
  )(x, W)


def _tc_scale(h, dinv_col):
  def body(h_ref, dv_ref, o_ref):
    o_ref[...] = h_ref[...] * dv_ref[...]

  return pl.pallas_call(
      body,
      grid=(N // _BN,),
      in_specs=[
          pl.BlockSpec((_BN, D), lambda i: (i, 0)),
          pl.BlockSpec((_BN, 1), lambda i: (i, 0)),
      ],
      out_specs=pl.BlockSpec((_BN, D), lambda i: (i, 0)),
      out_shape=jax.ShapeDtypeStruct((NPAD, D), jnp.float32),
  )(h, dinv_col)


def _tc_mid(acc, g, dinv_col, b, W):
  def body(a_ref, g_ref, dv_ref, b_ref, w_ref, o_ref):
    dv = dv_ref[...]
    tot = a_ref[0] + a_ref[1] + g_ref[...]
    z = jnp.maximum(tot * dv + b_ref[...], 0.0)
    h = jnp.dot(z, w_ref[...], preferred_element_type=jnp.float32)
    o_ref[...] = h * dv

  return pl.pallas_call(
      body,
      grid=(NPAD // _BR,),
      in_specs=[
          pl.BlockSpec((NC, _BR, D), lambda i: (0, i, 0)),
          pl.BlockSpec((_BR, D), lambda i: (i, 0)),
          pl.BlockSpec((_BR, 1), lambda i: (i, 0)),
          pl.BlockSpec((1, D), lambda i: (0, 0)),
          pl.BlockSpec((D, D), lambda i: (0, 0)),
      ],
      out_specs=pl.BlockSpec((_BR, D), lambda i: (i, 0)),
      out_shape=jax.ShapeDtypeStruct((NPAD, D), jnp.float32),
  )(acc, g, dinv_col, b, W)


def _tc_post(acc, g, dinv_col, b):
  def body(a_ref, g_ref, dv_ref, b_ref, o_ref):
    tot = a_ref[0] + a_ref[1] + g_ref[...]
    o_ref[...] = tot * dv_ref[...] + b_ref[...]

  return pl.pallas_call(
      body,
      grid=(N // _BN,),
      in_specs=[
          pl.BlockSpec((NC, _BN, D), lambda i: (0, i, 0)),
          pl.BlockSpec((_BN, D), lambda i: (i, 0)),
          pl.BlockSpec((_BN, 1), lambda i: (i, 0)),
          pl.BlockSpec((1, D), lambda i: (0, 0)),
      ],
      out_specs=pl.BlockSpec((_BN, D), lambda i: (i, 0)),
      out_shape=jax.ShapeDtypeStruct((N, D), jnp.float32),
  )(acc, g, dinv_col, b)


def kernel(x, edge_index, W1, b1, W2, b2):
  src = edge_index[0].astype(jnp.int32)
  dst = edge_index[1].astype(jnp.int32)
  E = src.shape[0]
  pad = EPAD - E
  fill = jnp.asarray(N + np.arange(pad, dtype=np.int32) % (NPAD - N))
  srcp = jnp.concatenate([src, fill])
  dstp = jnp.concatenate([dst, fill])
  src2 = srcp.reshape(EPAD // CH, CH)
  dst2 = dstp.reshape(EPAD // CH, CH)

  b1r = b1.reshape(1, D)
  b2r = b2.reshape(1, D)

  deg_p = _sc_hist(dstp).reshape(NW, NPAD // 128, 128)
  h1 = _tc_mm(x, W1)
  dinv = _tc_dinv(deg_p)
  dinv_col = dinv.reshape(NPAD, 1)

  g1 = _tc_scale(h1, dinv_col)
  acc1 = _sc_prop(g1, src2, dst2)
  g2 = _tc_mid(acc1, g1, dinv_col, b1r, W2)
  acc2 = _sc_prop(g2, src2, dst2)
  return _tc_post(acc2, g2, dinv_col, b2r)


# speedup vs baseline: 1.0159x; 1.0159x over previous
import dataclasses
import functools

import jax
import jax.numpy as jnp
import numpy as np
from jax import lax
from jax.experimental import pallas as pl
from jax.experimental.pallas import tpu as pltpu
from jax.experimental.pallas import tpu_sc as plsc

N = 10000
D = 128
NC = 2
NS = 16
LANES = 16
NW = NC * NS

NPAD = 10240
CH = 80
EPAD = NW * NPAD
CPW = EPAD // NW // CH
SLABC = 16
RPT = NPAD // NS

_mesh = plsc.VectorSubcoreMesh(
    core_axis_name="c", subcore_axis_name="s", num_cores=NC, num_subcores=NS
)

_sc_params = pltpu.CompilerParams()
if "needs_layout_passes" in pltpu.CompilerParams.__dataclass_fields__:
  _sc_params = dataclasses.replace(_sc_params, needs_layout_passes=False)


def _sc_hist(dst1d):
  @functools.partial(
      pl.kernel,
      out_type=jax.ShapeDtypeStruct((NW, NPAD), jnp.float32),
      mesh=_mesh,
      compiler_params=_sc_params,
      scratch_types=[
          pltpu.VMEM((CPW * CH,), jnp.int32),
          pltpu.VMEM((NPAD,), jnp.float32),
      ],
  )
  def k(d_hbm, o_hbm, didx, hist):
    c = lax.axis_index("c")
    s = lax.axis_index("s")
    wid = s * NC + c

    @pl.loop(0, NPAD, step=LANES)
    def _(i):
      hist.at[pl.ds(i, LANES)][...] = jnp.zeros((LANES,), jnp.float32)

    pltpu.sync_copy(d_hbm.at[pl.ds(wid * CPW * CH, CPW * CH)], didx)

    @pl.loop(0, CPW * CH, step=LANES)
    def _(t):
      idxv = didx.at[pl.ds(t, LANES)][...]
      plsc.addupdate_scatter(hist, [idxv], jnp.ones((LANES,), jnp.float32))

    pltpu.sync_copy(hist, o_hbm.at[wid])

  return k(dst1d)


def _sc_prop(g, src2, dst2):
  @functools.partial(
      pl.kernel,
      out_type=jax.ShapeDtypeStruct((NC, NPAD, D), jnp.float32),
      mesh=_mesh,
      compiler_params=_sc_params,
      scratch_types=[
          pltpu.VMEM((SLABC, CH), jnp.int32),
          pltpu.VMEM((SLABC, CH), jnp.int32),
          pltpu.VMEM((CH, D), jnp.float32),
          pltpu.VMEM((CH, D), jnp.float32),
          pltpu.VMEM_SHARED((NPAD, D), jnp.float32),
          pltpu.SemaphoreType.DMA,
          pltpu.SemaphoreType.DMA,
          pltpu.SemaphoreType.DMA,
      ],
  )
  def k(g_hbm, s_hbm, d_hbm, o_hbm, sidx, didx, rows0, rows1, acc,
        sem0, sem1, zsem):
    c = lax.axis_index("c")
    s = lax.axis_index("s")
    wid = c * NS + s
    base = wid * CPW

    pltpu.async_copy(s_hbm.at[pl.ds(base, SLABC)], sidx, sem0)
    pltpu.async_copy(d_hbm.at[pl.ds(base, SLABC)], didx, sem1)

    @pl.loop(0, CH)
    def _(r):
      @pl.loop(0, D, step=LANES)
      def _(l):
        rows0.at[r, pl.ds(l, LANES)][...] = jnp.zeros((LANES,), jnp.float32)

    @pl.loop(0, RPT, step=CH)
    def _(r0):
      pltpu.async_copy(rows0, acc.at[pl.ds(s * RPT + r0, CH)], zsem)

    @pl.loop(0, RPT, step=CH)
    def _(r0):
      pltpu.make_async_copy(
          rows0, acc.at[pl.ds(s * RPT + r0, CH)], zsem).wait()

    plsc.subcore_barrier()

    @pl.loop(0, CPW // SLABC)
    def _(sl):
      sbase = base + sl * SLABC

      @pl.when(sl > 0)
      def _():
        pltpu.async_copy(s_hbm.at[pl.ds(sbase, SLABC)], sidx, sem0)
        pltpu.async_copy(d_hbm.at[pl.ds(sbase, SLABC)], didx, sem1)

      pltpu.make_async_copy(s_hbm.at[pl.ds(sbase, SLABC)], sidx, sem0).wait()
      pltpu.make_async_copy(d_hbm.at[pl.ds(sbase, SLABC)], didx, sem1).wait()

      pltpu.async_copy(g_hbm.at[sidx.at[0]], rows0, sem0)

      @pl.loop(0, SLABC, step=2)
      def _(j):
        pltpu.async_copy(g_hbm.at[sidx.at[j + 1]], rows1, sem1)
        pltpu.make_async_copy(g_hbm.at[sidx.at[j]], rows0, sem0).wait()
        pltpu.sync_copy(rows0, acc.at[didx.at[j]], add=True)

        @pl.when(j + 2 < SLABC)
        def _():
          pltpu.async_copy(g_hbm.at[sidx.at[j + 2]], rows0, sem0)

        pltpu.make_async_copy(g_hbm.at[sidx.at[j + 1]], rows1, sem1).wait()
        pltpu.sync_copy(rows1, acc.at[didx.at[j + 1]], add=True)

    plsc.subcore_barrier()

    @pl.loop(0, RPT, step=CH)
    def _(r0):
      pltpu.async_copy(
          acc.at[pl.ds(s * RPT + r0, CH)],
          o_hbm.at[c, pl.ds(s * RPT + r0, CH)],
          zsem,
      )

    @pl.loop(0, RPT, step=CH)
    def _(r0):
      pltpu.make_async_copy(
          acc.at[pl.ds(s * RPT + r0, CH)],
          o_hbm.at[c, pl.ds(s * RPT + r0, CH)],
          zsem,
      ).wait()

  return k(g, src2, dst2)


_BR = 1024
_BN = 1000


def _tc_dinv(deg_p):
  def body(dp_ref, o_ref):
    deg = jnp.sum(dp_ref[...], axis=0) + 1.0
    o_ref[...] = lax.rsqrt(deg)

  return pl.pallas_call(
      body,
      out_shape=jax.ShapeDtypeStruct((NPAD // 128, 128), jnp.float32),
  )(deg_p)


def _tc_pre(x, W, dinv_col):
  def body(x_ref, w_ref, dv_ref, o_ref):
    h = jnp.dot(x_ref[...], w_ref[...], preferred_element_type=jnp.float32)
    o_ref[...] = h * dv_ref[...]

  return pl.pallas_call(
      body,
      grid=(N // _BN,),
      in_specs=[
          pl.BlockSpec((_BN, D), lambda i: (i, 0)),
          pl.BlockSpec((D, D), lambda i: (0, 0)),
          pl.BlockSpec((_BN, 1), lambda i: (i, 0)),
      ],
      out_specs=pl.BlockSpec((_BN, D), lambda i: (i, 0)),
      out_shape=jax.ShapeDtypeStruct((NPAD, D), jnp.float32),
  )(x, W, dinv_col)


def _tc_mid(acc, g, dinv_col, b, W):
  def body(a_ref, g_ref, dv_ref, b_ref, w_ref, o_ref):
    dv = dv_ref[...]
    tot = a_ref[0] + a_ref[1] + g_ref[...]
    z = jnp.maximum(tot * dv + b_ref[...], 0.0)
    h = jnp.dot(z, w_ref[...], preferred_element_type=jnp.float32)
    o_ref[...] = h * dv

  return pl.pallas_call(
      body,
      grid=(NPAD // _BR,),
      in_specs=[
          pl.BlockSpec((NC, _BR, D), lambda i: (0, i, 0)),
          pl.BlockSpec((_BR, D), lambda i: (i, 0)),
          pl.BlockSpec((_BR, 1), lambda i: (i, 0)),
          pl.BlockSpec((1, D), lambda i: (0, 0)),
          pl.BlockSpec((D, D), lambda i: (0, 0)),
      ],
      out_specs=pl.BlockSpec((_BR, D), lambda i: (i, 0)),
      out_shape=jax.ShapeDtypeStruct((NPAD, D), jnp.float32),
  )(acc, g, dinv_col, b, W)


def _tc_post(acc, g, dinv_col, b):
  def body(a_ref, g_ref, dv_ref, b_ref, o_ref):
    tot = a_ref[0] + a_ref[1] + g_ref[...]
    o_ref[...] = tot * dv_ref[...] + b_ref[...]

  return pl.pallas_call(
      body,
      grid=(N // _BN,),
      in_specs=[
          pl.BlockSpec((NC, _BN, D), lambda i: (0, i, 0)),
          pl.BlockSpec((_BN, D), lambda i: (i, 0)),
          pl.BlockSpec((_BN, 1), lambda i: (i, 0)),
          pl.BlockSpec((1, D), lambda i: (0, 0)),
      ],
      out_specs=pl.BlockSpec((_BN, D), lambda i: (i, 0)),
      out_shape=jax.ShapeDtypeStruct((N, D), jnp.float32),
  )(acc, g, dinv_col, b)


def kernel(x, edge_index, W1, b1, W2, b2):
  src = edge_index[0].astype(jnp.int32)
  dst = edge_index[1].astype(jnp.int32)
  E = src.shape[0]
  pad = EPAD - E
  fill = jnp.asarray(N + np.arange(pad, dtype=np.int32) % (NPAD - N))
  srcp = jnp.concatenate([src, fill])
  dstp = jnp.concatenate([dst, fill])
  src2 = srcp.reshape(EPAD // CH, CH)
  dst2 = dstp.reshape(EPAD // CH, CH)

  b1r = b1.reshape(1, D)
  b2r = b2.reshape(1, D)

  deg_p = _sc_hist(dstp).reshape(NW, NPAD // 128, 128)
  dinv = _tc_dinv(deg_p)
  dinv_col = dinv.reshape(NPAD, 1)

  g1 = _tc_pre(x, W1, dinv_col)
  acc1 = _sc_prop(g1, src2, dst2)
  g2 = _tc_mid(acc1, g1, dinv_col, b1r, W2)
  acc2 = _sc_prop(g2, src2, dst2)
  return _tc_post(acc2, g2, dinv_col, b2r)
